# baseline (speedup 1.0000x reference)
import functools

import jax
import jax.numpy as jnp
from jax import lax
from jax.experimental import pallas as pl
from jax.experimental.pallas import tpu as pltpu
from jax.experimental.pallas import tpu_sc as plsc

_NC = 2
_NS = 16
_NW = _NC * _NS
_L = 16

_B = 1280
_EVW = 16


def _lookup(ev16, vt_flat, ev_idx, vi_cols, *, n_rows, n_val, emb, n_cols):
    rows_per_w = n_rows // _NW
    n_chunks = rows_per_w // _B
    groups = _B // _L
    n_streams = _B // 128
    out_w = (n_cols + 1) * emb

    mesh = plsc.VectorSubcoreMesh(
        core_axis_name="c", subcore_axis_name="s",
        num_cores=_NC, num_subcores=_NS)

    @functools.partial(
        pl.kernel,
        out_type=jax.ShapeDtypeStruct((out_w, n_rows), jnp.float32),
        mesh=mesh,
        scratch_types=[
            pltpu.VMEM((n_cols * n_val * emb,), jnp.float32),
            pltpu.VMEM((_B,), jnp.int32),
            pltpu.VMEM((n_cols, _B), jnp.int32),
            pltpu.VMEM((_B, _EVW), jnp.float32),
            pltpu.VMEM((out_w, _B), jnp.float32),
            pltpu.SemaphoreType.DMA,
        ],
        compiler_params=pltpu.CompilerParams(
            needs_layout_passes=False, use_tc_tiling_on_sc=False),
    )
    def k(ev16_hbm, vt_hbm, ev_hbm, *rest):
        vi_hbms = rest[:n_cols]
        out_hbm = rest[n_cols]
        vt_v, ev_v, vi_v, stage_v, out_v, sem = rest[n_cols + 1:]
        wid = lax.axis_index("s") * _NC + lax.axis_index("c")
        base = wid * rows_per_w
        pltpu.sync_copy(vt_hbm, vt_v)

        def chunk(t, carry):
            rbase = base + t * _B
            pltpu.sync_copy(ev_hbm.at[pl.ds(rbase, _B)], ev_v)

            def fire(j, c2):
                pltpu.make_async_copy(
                    ev16_hbm.at[ev_v.at[pl.ds(j * 128, 128)]],
                    stage_v.at[pl.ds(j * 128, 128)], sem).start()
                return c2

            lax.fori_loop(0, n_streams, fire, 0)

            for c in range(n_cols):
                pltpu.sync_copy(vi_hbms[c].at[pl.ds(rbase, _B)], vi_v.at[c])

            def grp_val(g, c2):
                for c in range(n_cols):
                    iv = vi_v[c, pl.ds(g * _L, _L)]
                    iv3 = iv * emb + (c * n_val * emb)
                    for e in range(emb):
                        x = plsc.load_gather(vt_v, [iv3 + e])
                        out_v[(1 + c) * emb + e, pl.ds(g * _L, _L)] = x
                return c2

            lax.fori_loop(0, groups, grp_val, 0)

            def drain(j, c2):
                pltpu.make_async_copy(
                    ev16_hbm.at[ev_v.at[pl.ds(j * 128, 128)]],
                    stage_v.at[pl.ds(j * 128, 128)], sem).wait()
                return c2

            lax.fori_loop(0, n_streams, drain, 0)

            def grp_ev(g, c2):
                r16 = g * _L + lax.iota(jnp.int32, _L)
                for e in range(emb):
                    ecol = jnp.full((_L,), e, jnp.int32)
                    x = plsc.load_gather(stage_v, [r16, ecol])
                    out_v[e, pl.ds(g * _L, _L)] = x
                return c2

            lax.fori_loop(0, groups, grp_ev, 0)

            for r in range(out_w):
                pltpu.make_async_copy(
                    out_v.at[r], out_hbm.at[r, pl.ds(rbase, _B)], sem).start()
            for r in range(out_w):
                pltpu.make_async_copy(
                    out_v.at[r], out_hbm.at[r, pl.ds(rbase, _B)], sem).wait()
            return carry

        lax.fori_loop(0, n_chunks, chunk, 0)

    return k(ev16, vt_flat, ev_idx, *vi_cols)


def kernel(event_idx, value_idx, event_table, value_tables):
    n_rows = event_idx.shape[0]
    n_cols, n_val, emb = value_tables.shape
    ev16 = jnp.pad(event_table, ((0, 0), (0, _EVW - emb)))
    vt_flat = value_tables.reshape(n_cols * n_val * emb)
    vi32 = value_idx.astype(jnp.int32)
    vi_cols = [vi32[:, c] for c in range(n_cols)]
    out_t = _lookup(
        ev16, vt_flat,
        event_idx.astype(jnp.int32), vi_cols,
        n_rows=n_rows, n_val=n_val, emb=emb, n_cols=n_cols)
    return out_t.T


# speedup vs baseline: 4.1094x; 4.1094x over previous
import functools

import jax
import jax.numpy as jnp
from jax import lax
from jax.experimental import pallas as pl
from jax.experimental.pallas import tpu as pltpu
from jax.experimental.pallas import tpu_sc as plsc

_NC = 2
_NS = 16
_NW = _NC * _NS
_L = 16

_B = 1024
_EV_TILES = 12
_VAL_TILES = _NW - _EV_TILES


def _lookup(ev_cols, vt_flat, ev_idx, vi_cols, *, n_rows, n_val, emb, n_cols):
    out_w = (n_cols + 1) * emb
    kpad = 8 * ((out_w + 7) // 8)
    n_cb = n_rows // 128
    ev_per = _EV_TILES // emb
    ev_rows = n_rows // ev_per
    val_rows = n_rows // _VAL_TILES
    ev_chunks = ev_rows // _B
    val_chunks = val_rows // _B
    groups = _B // _L
    n_vw = n_cols * emb

    mesh = plsc.VectorSubcoreMesh(
        core_axis_name="c", subcore_axis_name="s",
        num_cores=_NC, num_subcores=_NS)

    @functools.partial(
        pl.kernel,
        out_type=jax.ShapeDtypeStruct((kpad // 8, n_cb, 8, 128), jnp.float32),
        mesh=mesh,
        scratch_types=[
            pltpu.VMEM((100001,), jnp.float32),
            pltpu.VMEM((n_cols, _B), jnp.int32),
            pltpu.VMEM((n_vw, _B // 128, 128), jnp.float32),
            pltpu.SemaphoreType.DMA,
        ],
        compiler_params=pltpu.CompilerParams(
            needs_layout_passes=False, use_tc_tiling_on_sc=False),
    )
    def k(*refs):
        ev_hbms = refs[:emb]
        vt_hbm = refs[emb]
        evi_hbm = refs[emb + 1]
        vi_hbms = refs[emb + 2:emb + 2 + n_cols]
        out_hbm = refs[emb + 2 + n_cols]
        tbl_v, idx_v, stage_v, sem = refs[emb + 3 + n_cols:]

        wid = lax.axis_index("s") * _NC + lax.axis_index("c")

        @pl.when(wid < _EV_TILES)
        def _event_tiles():
            comp = wid // ev_per
            quarter = wid % ev_per
            for c in range(emb):
                @pl.when(comp == c)
                def _():
                    pltpu.sync_copy(ev_hbms[c], tbl_v)

            def chunk(t, carry):
                rbase = quarter * ev_rows + t * _B
                pltpu.sync_copy(evi_hbm.at[pl.ds(rbase, _B)], idx_v.at[0])

                def grp4(g4, c2):
                    for u in range(4):
                        g = g4 * 4 + u
                        iv = idx_v[0, pl.ds(g * _L, _L)]
                        x = plsc.load_gather(tbl_v, [iv])
                        stage_v[0, g // 8, pl.ds((g % 8) * _L, _L)] = x
                    return c2

                lax.fori_loop(0, groups // 4, grp4, 0)
                cb0 = rbase // 128
                cp = pltpu.make_async_copy(
                    stage_v.at[0],
                    out_hbm.at[comp // 8, pl.ds(cb0, _B // 128), comp % 8],
                    sem)
                cp.start()
                cp.wait()
                return carry

            lax.fori_loop(0, ev_chunks, chunk, 0)

        @pl.when(wid >= _EV_TILES)
        def _value_tiles():
            vid = wid - _EV_TILES
            pltpu.sync_copy(vt_hbm, tbl_v.at[pl.ds(0, n_cols * n_val * emb)])

            def chunk(t, carry):
                rbase = vid * val_rows + t * _B
                for c in range(n_cols):
                    pltpu.sync_copy(vi_hbms[c].at[pl.ds(rbase, _B)],
                                    idx_v.at[c])

                def grp(g, c2):
                    for c in range(n_cols):
                        iv = idx_v[c, pl.ds(g * _L, _L)]
                        iv3 = iv * emb + (c * n_val * emb)
                        for e in range(emb):
                            x = plsc.load_gather(tbl_v, [iv3 + e])
                            stage_v[c * emb + e, g // 8,
                                    pl.ds((g % 8) * _L, _L)] = x
                    return c2

                lax.fori_loop(0, groups, grp, 0)
                cb0 = rbase // 128
                for w in range(n_vw):
                    kcomp = emb + w
                    pltpu.make_async_copy(
                        stage_v.at[w],
                        out_hbm.at[kcomp // 8, pl.ds(cb0, _B // 128),
                                   kcomp % 8],
                        sem).start()
                for w in range(n_vw):
                    kcomp = emb + w
                    pltpu.make_async_copy(
                        stage_v.at[w],
                        out_hbm.at[kcomp // 8, pl.ds(cb0, _B // 128),
                                   kcomp % 8],
                        sem).wait()
                return carry

            lax.fori_loop(0, val_chunks, chunk, 0)

    return k(*ev_cols, vt_flat, ev_idx, *vi_cols)


def kernel(event_idx, value_idx, event_table, value_tables):
    n_rows = event_idx.shape[0]
    n_cols, n_val, emb = value_tables.shape
    out_w = (n_cols + 1) * emb
    kpad = 8 * ((out_w + 7) // 8)
    ev_cols = [event_table[:, c] for c in range(emb)]
    vt_flat = value_tables.reshape(n_cols * n_val * emb)
    vi32 = value_idx.astype(jnp.int32)
    vi_cols = [vi32[:, c] for c in range(n_cols)]
    out4 = _lookup(
        ev_cols, vt_flat,
        event_idx.astype(jnp.int32), vi_cols,
        n_rows=n_rows, n_val=n_val, emb=emb, n_cols=n_cols)
    out = out4.transpose(1, 3, 0, 2).reshape(n_rows, kpad)
    return out[:, :out_w]


# speedup vs baseline: 4.8967x; 1.1916x over previous
import functools

import jax
import jax.numpy as jnp
from jax import lax
from jax.experimental import pallas as pl
from jax.experimental.pallas import tpu as pltpu
from jax.experimental.pallas import tpu_sc as plsc

_NC = 2
_NS = 16
_NW = _NC * _NS
_L = 16

_B = 1024
_EV_TILES = 12
_VAL_TILES = _NW - _EV_TILES


def _lookup(ev_cols, vt_flat, ev_idx, vi_cols, *, n_rows, n_val, emb, n_cols):
    out_w = (n_cols + 1) * emb
    kpad = 8 * ((out_w + 7) // 8)
    n_cb = n_rows // 128
    ev_per = _EV_TILES // emb
    ev_rows = n_rows // ev_per
    val_rows = n_rows // _VAL_TILES
    ev_chunks = ev_rows // _B
    val_chunks = val_rows // _B
    groups = _B // _L
    n_vw = n_cols * emb

    mesh = plsc.VectorSubcoreMesh(
        core_axis_name="c", subcore_axis_name="s",
        num_cores=_NC, num_subcores=_NS)

    @functools.partial(
        pl.kernel,
        out_type=jax.ShapeDtypeStruct((kpad // 8, n_cb, 8, 128), jnp.float32),
        mesh=mesh,
        scratch_types=[
            pltpu.VMEM((100001,), jnp.float32),
            pltpu.VMEM((n_cols * _B,), jnp.int32),
            pltpu.VMEM((n_vw * (_B // 128), 128), jnp.float32),
            pltpu.SemaphoreType.DMA,
        ],
        compiler_params=pltpu.CompilerParams(
            needs_layout_passes=False, use_tc_tiling_on_sc=False),
    )
    def k(*refs):
        ev_hbms = refs[:emb]
        vt_hbm = refs[emb]
        evi_hbm = refs[emb + 1]
        vi_hbms = refs[emb + 2:emb + 2 + n_cols]
        out_hbm = refs[emb + 2 + n_cols]
        tbl_v, idx_v, stage_v, sem = refs[emb + 3 + n_cols:]

        wid = lax.axis_index("s") * _NC + lax.axis_index("c")

        @pl.when(wid < _EV_TILES)
        def _event_tiles():
            comp = wid // ev_per
            quarter = wid % ev_per
            for c in range(emb):
                @pl.when(comp == c)
                def _():
                    pltpu.sync_copy(ev_hbms[c], tbl_v)

            b_ev = n_cols * _B
            ev_groups = b_ev // _L

            def chunk(t, carry):
                rbase = quarter * ev_rows + t * b_ev
                pltpu.sync_copy(evi_hbm.at[pl.ds(rbase, b_ev)], idx_v)

                def grp4(g4, c2):
                    for u in range(4):
                        g = g4 * 4 + u
                        iv = idx_v[pl.ds(g * _L, _L)]
                        x = plsc.load_gather(tbl_v, [iv])
                        stage_v[g // 8, pl.ds((g % 8) * _L, _L)] = x
                    return c2

                lax.fori_loop(0, ev_groups // 4, grp4, 0)
                cb0 = rbase // 128
                cp = pltpu.make_async_copy(
                    stage_v.at[pl.ds(0, b_ev // 128)],
                    out_hbm.at[comp // 8, pl.ds(cb0, b_ev // 128), comp % 8],
                    sem)
                cp.start()
                cp.wait()
                return carry

            lax.fori_loop(0, ev_rows // b_ev, chunk, 0)

        @pl.when(wid >= _EV_TILES)
        def _value_tiles():
            vid = wid - _EV_TILES
            pltpu.sync_copy(vt_hbm, tbl_v.at[pl.ds(0, n_cols * n_val * emb)])

            nb = _B // 128

            def chunk(t, carry):
                rbase = vid * val_rows + t * _B
                for c in range(n_cols):
                    pltpu.make_async_copy(
                        vi_hbms[c].at[pl.ds(rbase, _B)],
                        idx_v.at[pl.ds(c * _B, _B)], sem).start()
                for c in range(n_cols):
                    pltpu.make_async_copy(
                        vi_hbms[c].at[pl.ds(rbase, _B)],
                        idx_v.at[pl.ds(c * _B, _B)], sem).wait()

                def grp(g, c2):
                    for c in range(n_cols):
                        iv = idx_v[pl.ds(c * _B + g * _L, _L)]
                        iv3 = iv * emb + (c * n_val * emb)
                        for e in range(emb):
                            x = plsc.load_gather(tbl_v, [iv3 + e])
                            stage_v[(c * emb + e) * nb + g // 8,
                                    pl.ds((g % 8) * _L, _L)] = x
                    return c2

                lax.fori_loop(0, groups, grp, 0)
                cb0 = rbase // 128
                for w in range(n_vw):
                    kcomp = emb + w
                    pltpu.make_async_copy(
                        stage_v.at[pl.ds(w * nb, nb)],
                        out_hbm.at[kcomp // 8, pl.ds(cb0, nb), kcomp % 8],
                        sem).start()
                for w in range(n_vw):
                    kcomp = emb + w
                    pltpu.make_async_copy(
                        stage_v.at[pl.ds(w * nb, nb)],
                        out_hbm.at[kcomp // 8, pl.ds(cb0, nb), kcomp % 8],
                        sem).wait()
                return carry

            lax.fori_loop(0, val_chunks, chunk, 0)

    return k(*ev_cols, vt_flat, ev_idx, *vi_cols)


def kernel(event_idx, value_idx, event_table, value_tables):
    n_rows = event_idx.shape[0]
    n_cols, n_val, emb = value_tables.shape
    out_w = (n_cols + 1) * emb
    kpad = 8 * ((out_w + 7) // 8)
    ev_cols = [event_table[:, c] for c in range(emb)]
    vt_flat = value_tables.reshape(n_cols * n_val * emb)
    vi32 = value_idx.astype(jnp.int32)
    vi_cols = [vi32[:, c] for c in range(n_cols)]
    out4 = _lookup(
        ev_cols, vt_flat,
        event_idx.astype(jnp.int32), vi_cols,
        n_rows=n_rows, n_val=n_val, emb=emb, n_cols=n_cols)
    out = out4.transpose(1, 3, 0, 2).reshape(n_rows, kpad)
    return out[:, :out_w]


# speedup vs baseline: 5.0175x; 1.0247x over previous
import functools

import jax
import jax.numpy as jnp
from jax import lax
from jax.experimental import pallas as pl
from jax.experimental.pallas import tpu as pltpu
from jax.experimental.pallas import tpu_sc as plsc

_NC = 2
_NS = 16
_NW = _NC * _NS
_L = 16

_B = 1280
_EV_TILES = 12
_VAL_TILES = _NW - _EV_TILES


def _lookup(ev_cols, vt_flat, ev_idx, vi_cols, *, n_rows, n_val, emb, n_cols):
    out_w = (n_cols + 1) * emb
    kpad = 8 * ((out_w + 7) // 8)
    n_cb = n_rows // 128
    ev_per = _EV_TILES // emb
    ev_rows = n_rows // ev_per
    val_rows = n_rows // _VAL_TILES
    ev_chunks = ev_rows // _B
    val_chunks = val_rows // _B
    groups = _B // _L
    n_vw = n_cols * emb

    mesh = plsc.VectorSubcoreMesh(
        core_axis_name="c", subcore_axis_name="s",
        num_cores=_NC, num_subcores=_NS)

    @functools.partial(
        pl.kernel,
        out_type=jax.ShapeDtypeStruct((kpad // 8, n_cb, 8, 128), jnp.float32),
        mesh=mesh,
        scratch_types=[
            pltpu.VMEM((100001,), jnp.float32),
            pltpu.VMEM((n_cols * _B,), jnp.int32),
            pltpu.VMEM((n_vw * (_B // 128), 128), jnp.float32),
            pltpu.SemaphoreType.DMA,
        ],
        compiler_params=pltpu.CompilerParams(
            needs_layout_passes=False, use_tc_tiling_on_sc=False),
    )
    def k(*refs):
        ev_hbms = refs[:emb]
        vt_hbm = refs[emb]
        evi_hbm = refs[emb + 1]
        vi_hbms = refs[emb + 2:emb + 2 + n_cols]
        out_hbm = refs[emb + 2 + n_cols]
        tbl_v, idx_v, stage_v, sem = refs[emb + 3 + n_cols:]

        wid = lax.axis_index("s") * _NC + lax.axis_index("c")

        @pl.when(wid < _EV_TILES)
        def _event_tiles():
            comp = wid // ev_per
            quarter = wid % ev_per
            for c in range(emb):
                @pl.when(comp == c)
                def _():
                    pltpu.sync_copy(ev_hbms[c], tbl_v)

            b_ev = n_cols * _B
            ev_groups = b_ev // _L

            def chunk(t, carry):
                rbase = quarter * ev_rows + t * b_ev
                pltpu.sync_copy(evi_hbm.at[pl.ds(rbase, b_ev)], idx_v)

                def grp4(g4, c2):
                    for u in range(4):
                        g = g4 * 4 + u
                        iv = idx_v[pl.ds(g * _L, _L)]
                        x = plsc.load_gather(tbl_v, [iv])
                        stage_v[g // 8, pl.ds((g % 8) * _L, _L)] = x
                    return c2

                lax.fori_loop(0, ev_groups // 4, grp4, 0)
                cb0 = rbase // 128
                cp = pltpu.make_async_copy(
                    stage_v.at[pl.ds(0, b_ev // 128)],
                    out_hbm.at[comp // 8, pl.ds(cb0, b_ev // 128), comp % 8],
                    sem)
                cp.start()
                cp.wait()
                return carry

            lax.fori_loop(0, ev_rows // b_ev, chunk, 0)

        @pl.when(wid >= _EV_TILES)
        def _value_tiles():
            vid = wid - _EV_TILES
            pltpu.sync_copy(vt_hbm, tbl_v.at[pl.ds(0, n_cols * n_val * emb)])

            nb = _B // 128

            def chunk(t, carry):
                rbase = vid * val_rows + t * _B
                for c in range(n_cols):
                    pltpu.make_async_copy(
                        vi_hbms[c].at[pl.ds(rbase, _B)],
                        idx_v.at[pl.ds(c * _B, _B)], sem).start()
                for c in range(n_cols):
                    pltpu.make_async_copy(
                        vi_hbms[c].at[pl.ds(rbase, _B)],
                        idx_v.at[pl.ds(c * _B, _B)], sem).wait()

                def grp(g2, c2):
                    for u in range(2):
                        g = g2 * 2 + u
                        for c in range(n_cols):
                            iv = idx_v[pl.ds(c * _B + g * _L, _L)]
                            iv3 = iv * emb + (c * n_val * emb)
                            for e in range(emb):
                                x = plsc.load_gather(tbl_v, [iv3 + e])
                                stage_v[(c * emb + e) * nb + g // 8,
                                        pl.ds((g % 8) * _L, _L)] = x
                    return c2

                lax.fori_loop(0, groups // 2, grp, 0)
                cb0 = rbase // 128
                for w in range(n_vw):
                    kcomp = emb + w
                    pltpu.make_async_copy(
                        stage_v.at[pl.ds(w * nb, nb)],
                        out_hbm.at[kcomp // 8, pl.ds(cb0, nb), kcomp % 8],
                        sem).start()
                for w in range(n_vw):
                    kcomp = emb + w
                    pltpu.make_async_copy(
                        stage_v.at[pl.ds(w * nb, nb)],
                        out_hbm.at[kcomp // 8, pl.ds(cb0, nb), kcomp % 8],
                        sem).wait()
                return carry

            lax.fori_loop(0, val_chunks, chunk, 0)

    return k(*ev_cols, vt_flat, ev_idx, *vi_cols)


def kernel(event_idx, value_idx, event_table, value_tables):
    n_rows = event_idx.shape[0]
    n_cols, n_val, emb = value_tables.shape
    out_w = (n_cols + 1) * emb
    kpad = 8 * ((out_w + 7) // 8)
    ev_cols = [event_table[:, c] for c in range(emb)]
    vt_flat = value_tables.reshape(n_cols * n_val * emb)
    vi32 = value_idx.astype(jnp.int32)
    vi_cols = [vi32[:, c] for c in range(n_cols)]
    out4 = _lookup(
        ev_cols, vt_flat,
        event_idx.astype(jnp.int32), vi_cols,
        n_rows=n_rows, n_val=n_val, emb=emb, n_cols=n_cols)
    out = out4.transpose(1, 3, 0, 2).reshape(n_rows, kpad)
    return out[:, :out_w]
